# baseline (speedup 1.0000x reference)
import functools

import jax
import jax.numpy as jnp
from jax import lax
from jax.experimental import pallas as pl
from jax.experimental.pallas import tpu as pltpu
from jax.experimental.pallas import tpu_sc as plsc

_NW = 32
_L = 16


def _gather_kernel(batches_per_w, hist, idx_hbm, wide_hbm, out_hbm,
                   idxb0, idxb1, rows0, rows1, sel0, sel1,
                   si0, si1, sg0, sg1, so0, so1):
    wid = lax.axis_index("s") * 2 + lax.axis_index("c")
    b_base = wid * batches_per_w

    splits = [(0, 128), (128, hist - 128)] if hist > 128 else [(0, hist)]

    def stage_idx(g, idxb, sem):
        pltpu.async_copy(idx_hbm.at[b_base + g], idxb, sem)

    def drain_idx(idxb, sem):
        pltpu.make_async_copy(idx_hbm.at[b_base], idxb, sem).wait()

    def fire_group(idxb, rows, sem):
        for (off, ln) in splits:
            pltpu.async_copy(
                wide_hbm.at[idxb.at[pl.ds(off, ln)]],
                rows.at[pl.ds(off, ln)],
                sem)

    def compact(rows, sel):
        def row(r, _):
            for k in range(64 // _L):
                sel[r, pl.ds(k * _L, _L)] = rows[r, pl.ds(k * _L, _L)]
            return 0

        lax.fori_loop(0, hist, row, 0)

    def fire_out(g, sel, sem):
        pltpu.async_copy(sel, out_hbm.at[b_base + g], sem)

    def drain_gather(rows, sem):
        pltpu.make_async_copy(
            wide_hbm.at[pl.ds(0, hist)], rows, sem).wait()

    def drain_out(sel, sem):
        pltpu.make_async_copy(sel, out_hbm.at[b_base], sem).wait()

    pltpu.sync_copy(idx_hbm.at[b_base], idxb0)
    fire_group(idxb0, rows0, sg0)
    stage_idx(1, idxb1, si1)
    npairs = batches_per_w // 2

    def body(t, _):
        a = 2 * t
        more = t < npairs - 1

        @pl.when(t > 0)
        def _():
            drain_out(sel1, so1)

        drain_idx(idxb1, si1)
        fire_group(idxb1, rows1, sg1)
        drain_gather(rows0, sg0)

        @pl.when(more)
        def _():
            stage_idx(a + 2, idxb0, si0)

        compact(rows0, sel0)
        fire_out(a, sel0, so0)
        drain_out(sel0, so0)

        @pl.when(more)
        def _():
            drain_idx(idxb0, si0)
            fire_group(idxb0, rows0, sg0)

        drain_gather(rows1, sg1)

        @pl.when(more)
        def _():
            stage_idx(a + 3, idxb1, si1)

        compact(rows1, sel1)
        fire_out(a + 1, sel1, so1)
        return 0

    lax.fori_loop(0, npairs, body, 0)
    drain_out(sel1, so1)


def kernel(paragraph_variable, embedding_table):
    B, H = paragraph_variable.shape
    V, D = embedding_table.shape
    batches_per_w = B // _NW

    wide = lax.dynamic_update_slice(
        jnp.zeros((V, 2 * D), jnp.float32), embedding_table, (0, 0))

    mesh = plsc.VectorSubcoreMesh(core_axis_name="c", subcore_axis_name="s")
    gather = pl.kernel(
        functools.partial(_gather_kernel, batches_per_w, H),
        mesh=mesh,
        out_type=jax.ShapeDtypeStruct((B, H, D), jnp.float32),
        scratch_types=[
            pltpu.VMEM((H,), jnp.int32),
            pltpu.VMEM((H,), jnp.int32),
            pltpu.VMEM((H, 2 * D), jnp.float32),
            pltpu.VMEM((H, 2 * D), jnp.float32),
            pltpu.VMEM((H, D), jnp.float32),
            pltpu.VMEM((H, D), jnp.float32),
            pltpu.SemaphoreType.DMA,
            pltpu.SemaphoreType.DMA,
            pltpu.SemaphoreType.DMA,
            pltpu.SemaphoreType.DMA,
            pltpu.SemaphoreType.DMA,
            pltpu.SemaphoreType.DMA,
        ],
    )
    return gather(paragraph_variable, wide)
